# baseline (speedup 1.0000x reference)
import functools

import jax
import jax.numpy as jnp
from jax import lax
from jax.experimental import pallas as pl
from jax.experimental.pallas import tpu as pltpu
from jax.experimental.pallas import tpu_sc as plsc

_B = 8
_V = 1000000
_C = 65536
_NC = 16
_G = 1024
_NG = 977
_NEG = -jnp.inf
_IBIG = 2**30

_WG = 31
_WL = _WG * _G


def _sc_group_max(buf, obuf, b, g_chunk, oslot, n_vregs):
    accs = [buf[b, pl.ds(g_chunk * _G + k * 16, 16)] for k in range(4)]
    for k in range(4, n_vregs):
        accs[k % 4] = jnp.maximum(accs[k % 4],
                                  buf[b, pl.ds(g_chunk * _G + k * 16, 16)])
    obuf[pl.ds(oslot * 16, 16)] = jnp.maximum(
        jnp.maximum(accs[0], accs[1]), jnp.maximum(accs[2], accs[3]))


def _sc_filter_body(x_hbm, out_hbm, buf, obuf):
    wid = lax.axis_index("s") * 2 + lax.axis_index("c")
    start = wid * _WL

    def make_loop(goff, ng):
        def body(t, carry):
            b = t // ng
            g = t - b * ng
            _sc_group_max(buf, obuf, b, g, b * 32 + goff + g, 64)
            return carry
        return body

    @pl.when(wid < 31)
    def _main():
        for ch, (goff, ng) in enumerate([(0, 8), (8, 8), (16, 8), (24, 7)]):
            pltpu.sync_copy(x_hbm.at[:, pl.ds(start + goff * _G, ng * _G)],
                            buf.at[:, pl.ds(0, ng * _G)])
            lax.fori_loop(0, 8 * ng, make_loop(goff, ng), 0)

    @pl.when(wid == 31)
    def _tail():
        pltpu.sync_copy(x_hbm.at[:, pl.ds(start, 8 * _G)], buf)
        lax.fori_loop(0, 64, make_loop(0, 8), 0)
        pltpu.sync_copy(x_hbm.at[:, pl.ds(start + 8 * _G, 7 * _G)],
                        buf.at[:, pl.ds(0, 7 * _G)])
        lax.fori_loop(0, 56, make_loop(8, 7), 0)

    for b in range(_B):
        pltpu.sync_copy(obuf.at[pl.ds(b * 512, 512)],
                        out_hbm.at[pl.ds((b * 32 + wid) * 512, 512)])


_sc_filter = functools.partial(
    pl.kernel,
    out_type=jax.ShapeDtypeStruct((_B * 32 * 512,), jnp.float32),
    mesh=plsc.VectorSubcoreMesh(core_axis_name="c", subcore_axis_name="s",
                                num_cores=2, num_subcores=16),
    scratch_types=[
        pltpu.VMEM((_B, 8 * _G), jnp.float32),
        pltpu.VMEM((_B * 32 * 16,), jnp.float32),
    ],
)(_sc_filter_body)


def _treemax(xs):
    xs = list(xs)
    while len(xs) > 1:
        nxt = [jnp.maximum(xs[k], xs[k + 1]) for k in range(0, len(xs) - 1, 2)]
        if len(xs) % 2:
            nxt.append(xs[-1])
        xs = nxt
    return xs[0]


def _scan_chunk(x, m_scr, s_scr):
    sl = [x[:, k * 128:(k + 1) * 128] for k in range(_C // 128)]
    gfold = _treemax(sl)
    m_c = jnp.max(gfold, axis=1, keepdims=True)
    m_old = m_scr[...]
    m_new = jnp.maximum(m_old, m_c)
    mb = m_new[:, 0:1]

    accs = [jnp.zeros((_B, 128), jnp.float32) for _ in range(4)]
    for k in range(_C // 128):
        accs[k % 4] = accs[k % 4] + jnp.exp(sl[k] - mb)
    acc = (accs[0] + accs[1]) + (accs[2] + accs[3])

    s_scr[...] = s_scr[...] * jnp.exp(m_old - m_new) + acc
    m_scr[...] = m_new


def _scan_body(x_ref, stats_ref, m_scr, s_scr):
    i = pl.program_id(0)

    @pl.when(i == 0)
    def _init():
        m_scr[...] = jnp.full((_B, 128), _NEG, jnp.float32)
        s_scr[...] = jnp.zeros((_B, 128), jnp.float32)

    @pl.when(i < _NC - 1)
    def _full():
        _scan_chunk(x_ref[...], m_scr, s_scr)

    @pl.when(i == _NC - 1)
    def _edge():
        tail = _V - (_NC - 1) * _C
        liota = jax.lax.broadcasted_iota(jnp.int32, (_B, _C), 1)
        x = jnp.where(liota < tail, x_ref[...], _NEG)
        _scan_chunk(x, m_scr, s_scr)
        l128 = jax.lax.broadcasted_iota(jnp.int32, (_B, 128), 1)
        s_b = jnp.sum(s_scr[...], axis=1, keepdims=True)
        stats_ref[...] = jnp.where(l128 == 0, m_scr[...],
                                   jnp.where(l128 == 1, s_b, 0.0))


def _select_body(gm_ref, xt_ref, ids_ref):
    r = gm_ref[...]
    gmax = jnp.max(r, axis=3)
    lt = jax.lax.broadcasted_iota(jnp.int32, (_B, _G), 1)
    tmax = jnp.max(jnp.where(lt < _V - (_NG - 1) * _G, xt_ref[...], _NEG),
                   axis=1, keepdims=True)
    w_i = jax.lax.broadcasted_iota(jnp.int32, (32, 32), 0)
    gs = jax.lax.broadcasted_iota(jnp.int32, (32, 32), 1)
    gid = w_i * _WG + gs
    valid = (gs < _WG) & (gid < _NG - 1)
    tail_slot = (w_i == 31) & (gs == 15)
    lane8 = jax.lax.broadcasted_iota(jnp.int32, (1, _B), 1)
    id_rows = []
    for bb in range(_B):
        sub = jnp.where(valid, gmax[bb], _NEG)
        sub = jnp.where(tail_slot, tmax[bb:bb + 1, 0:1], sub)
        idr = jnp.zeros((1, _B), jnp.int32)
        for j in range(_B):
            v = jnp.max(sub)
            sel = jnp.min(jnp.where(sub >= v, gid, _IBIG))
            idr = jnp.where(lane8 == j, sel, idr)
            sub = jnp.where(gid == sel, _NEG, sub)
        id_rows.append(idr)
    ids_ref[...] = jnp.concatenate(id_rows, axis=0)


def _merge_body(ids_ref, *refs):
    xrefs = refs[:_B * _B]
    stats_ref, energy_ref, hid_ref = refs[_B * _B:_B * _B + 3]
    act_ref, logp_ref, eng_ref, tok_ref, hid_out_ref = refs[_B * _B + 3:]

    li = jax.lax.broadcasted_iota(jnp.int32, (1, _G), 1)
    rows = []
    lidx_rows = []
    for b in range(_B):
        rows.append(jnp.concatenate(
            [xrefs[b * _B + j][b:b + 1, :] for j in range(_B)], axis=1))
        lidx_rows.append(jnp.concatenate(
            [ids_ref[b * _B + j] * _G + li for j in range(_B)], axis=1))
    cand = jnp.concatenate(rows, axis=0)
    lidx = jnp.concatenate(lidx_rows, axis=0)
    cand = jnp.where(lidx < _V, cand, _NEG)

    lane8 = jax.lax.broadcasted_iota(jnp.int32, (1, _B), 1)
    topv = jnp.zeros((_B, _B), jnp.float32)
    topi = jnp.zeros((_B, _B), jnp.int32)
    for j in range(_B):
        v = jnp.max(cand, axis=1, keepdims=True)
        sel = jnp.min(jnp.where(cand >= v, lidx, _IBIG),
                      axis=1, keepdims=True)
        topv = jnp.where(lane8 == j, v, topv)
        topi = jnp.where(lane8 == j, sel, topi)
        cand = jnp.where(lidx == sel, _NEG, cand)

    stats = stats_ref[...]
    m = stats[:, 0:1]
    s = stats[:, 1:2]
    energy = energy_ref[...]
    cand_logp = (topv - m) - jnp.log(s)
    action = energy - cand_logp

    cidx = (jax.lax.broadcasted_iota(jnp.int32, (_B, _B), 0) * _B
            + jax.lax.broadcasted_iota(jnp.int32, (_B, _B), 1))
    row8 = jax.lax.broadcasted_iota(jnp.int32, (_B, _B), 0)
    col8 = jax.lax.broadcasted_iota(jnp.int32, (_B, _B), 1)

    a_acc = jnp.zeros((1, _B), jnp.float32)
    lp_acc = jnp.zeros((1, _B), jnp.float32)
    en_acc = jnp.zeros((1, _B), jnp.float32)
    tk_acc = jnp.zeros((1, _B), jnp.int32)
    onehot = jnp.zeros((_B, _B), jnp.float32)
    energy_b = jnp.broadcast_to(energy, (_B, _B))

    aw = action
    for j in range(_B):
        a = jnp.min(aw)
        sel = jnp.min(jnp.where(aw <= a, cidx, _IBIG))
        hit = cidx == sel
        a_acc = jnp.where(lane8 == j, a, a_acc)
        lp_acc = jnp.where(lane8 == j,
                           jnp.sum(jnp.where(hit, cand_logp, 0.0)), lp_acc)
        en_acc = jnp.where(lane8 == j,
                           jnp.sum(jnp.where(hit, energy_b, 0.0)), en_acc)
        tk_acc = jnp.where(lane8 == j, jnp.sum(jnp.where(hit, topi, 0)),
                           tk_acc)
        beam = sel // _B
        onehot = onehot + jnp.where((row8 == j) & (col8 == beam), 1.0, 0.0)
        aw = jnp.where(hit, jnp.inf, aw)

    act_ref[...] = a_acc
    logp_ref[...] = lp_acc
    eng_ref[...] = en_acc
    tok_ref[...] = tk_acc
    hid_out_ref[...] = jax.lax.dot(onehot, hid_ref[...],
                                   preferred_element_type=jnp.float32)


@functools.partial(jax.jit, static_argnames=("interpret",))
def _impl(next_token_logits, next_hidden_states, energies, interpret=False):
    gm_flat = _sc_filter(next_token_logits)
    gm4 = gm_flat.reshape(_B, 32, 32, 16)

    stats = pl.pallas_call(
        _scan_body,
        grid=(_NC,),
        in_specs=[pl.BlockSpec((_B, _C), lambda i: (0, i))],
        out_specs=pl.BlockSpec((_B, 128), lambda i: (0, 0)),
        out_shape=jax.ShapeDtypeStruct((_B, 128), jnp.float32),
        scratch_shapes=[
            pltpu.VMEM((_B, 128), jnp.float32),
            pltpu.VMEM((_B, 128), jnp.float32),
        ],
        interpret=interpret,
    )(next_token_logits)

    ids = pl.pallas_call(
        _select_body,
        grid=(1,),
        in_specs=[
            pl.BlockSpec((_B, 32, 32, 16), lambda i: (0, 0, 0, 0)),
            pl.BlockSpec((_B, _G), lambda i: (0, _NG - 1)),
        ],
        out_specs=pl.BlockSpec((_B, _B), lambda i: (0, 0)),
        out_shape=jax.ShapeDtypeStruct((_B, _B), jnp.int32),
        interpret=interpret,
    )(gm4, next_token_logits)

    def _xmap(r):
        return lambda i, ids_pf: (0, ids_pf[r])

    grid_spec = pltpu.PrefetchScalarGridSpec(
        num_scalar_prefetch=1,
        grid=(1,),
        in_specs=(
            [pl.BlockSpec((_B, _G), _xmap(r)) for r in range(_B * _B)]
            + [
                pl.BlockSpec((_B, 128), lambda i, ids_pf: (0, 0)),
                pl.BlockSpec((_B, 1), lambda i, ids_pf: (0, 0)),
                pl.BlockSpec((_B, 4096), lambda i, ids_pf: (0, 0)),
            ]),
        out_specs=[
            pl.BlockSpec((1, _B), lambda i, ids_pf: (0, 0)),
            pl.BlockSpec((1, _B), lambda i, ids_pf: (0, 0)),
            pl.BlockSpec((1, _B), lambda i, ids_pf: (0, 0)),
            pl.BlockSpec((1, _B), lambda i, ids_pf: (0, 0)),
            pl.BlockSpec((_B, 4096), lambda i, ids_pf: (0, 0)),
        ],
    )
    act, logp, eng, tok, hid = pl.pallas_call(
        _merge_body,
        grid_spec=grid_spec,
        out_shape=[
            jax.ShapeDtypeStruct((1, _B), jnp.float32),
            jax.ShapeDtypeStruct((1, _B), jnp.float32),
            jax.ShapeDtypeStruct((1, _B), jnp.float32),
            jax.ShapeDtypeStruct((1, _B), jnp.int32),
            jax.ShapeDtypeStruct((_B, 4096), jnp.float32),
        ],
        interpret=interpret,
    )(ids.reshape(_B * _B), *([next_token_logits] * (_B * _B)),
      stats, energies.reshape(_B, 1), next_hidden_states)

    return (act.reshape(_B), logp.reshape(_B), eng.reshape(_B),
            tok.reshape(_B), hid)


def kernel(next_token_logits, next_hidden_states, energies):
    return _impl(next_token_logits, next_hidden_states, energies)


# speedup vs baseline: 1.0585x; 1.0585x over previous
import functools

import jax
import jax.numpy as jnp
from jax import lax
from jax.experimental import pallas as pl
from jax.experimental.pallas import tpu as pltpu
from jax.experimental.pallas import tpu_sc as plsc

_B = 8
_V = 1000000
_C = 65536
_NC = 16
_G = 1024
_NG = 977
_NEG = -jnp.inf
_IBIG = 2**30

_WG = 31
_WL = _WG * _G


def _sc_group_max(buf, obuf, b, g_chunk, oslot, n_vregs):
    accs = [buf[b, pl.ds(g_chunk * _G + k * 16, 16)] for k in range(4)]
    for k in range(4, n_vregs):
        accs[k % 4] = jnp.maximum(accs[k % 4],
                                  buf[b, pl.ds(g_chunk * _G + k * 16, 16)])
    obuf[pl.ds(oslot * 16, 16)] = jnp.maximum(
        jnp.maximum(accs[0], accs[1]), jnp.maximum(accs[2], accs[3]))


def _sc_filter_body(x_hbm, out_hbm, buf0, buf1, obuf, sem0, sem1):
    wid = lax.axis_index("s") * 2 + lax.axis_index("c")
    start = wid * _WL
    bufs = [buf0, buf1]
    sems = [sem0, sem1]

    def make_loop(buf, goff, ng):
        def body(t, carry):
            b = t // ng
            g = t - b * ng
            _sc_group_max(buf, obuf, b, g, b * 32 + goff + g, 64)
            return carry
        return body

    def run(sizes):
        offs = [sum(sizes[:k]) for k in range(len(sizes))]
        n = len(sizes)

        def issue(ch):
            slot = ch % 2
            return pltpu.async_copy(
                x_hbm.at[:, pl.ds(start + offs[ch] * _G, sizes[ch] * _G)],
                bufs[slot].at[:, pl.ds(0, sizes[ch] * _G)], sems[slot])

        copies = {}
        for ch in range(min(2, n)):
            copies[ch] = issue(ch)
        for ch in range(n):
            copies[ch].wait()
            lax.fori_loop(0, 8 * sizes[ch],
                          make_loop(bufs[ch % 2], offs[ch], sizes[ch]), 0)
            if ch + 2 < n:
                copies[ch + 2] = issue(ch + 2)

    @pl.when(wid < 31)
    def _main():
        run([6, 6, 6, 6, 6, 1])

    @pl.when(wid == 31)
    def _tail():
        run([6, 6, 3])

    for b in range(_B):
        pltpu.sync_copy(obuf.at[pl.ds(b * 512, 512)],
                        out_hbm.at[pl.ds((b * 32 + wid) * 512, 512)])


_sc_filter = functools.partial(
    pl.kernel,
    out_type=jax.ShapeDtypeStruct((_B * 32 * 512,), jnp.float32),
    mesh=plsc.VectorSubcoreMesh(core_axis_name="c", subcore_axis_name="s",
                                num_cores=2, num_subcores=16),
    scratch_types=[
        pltpu.VMEM((_B, 6 * _G), jnp.float32),
        pltpu.VMEM((_B, 6 * _G), jnp.float32),
        pltpu.VMEM((_B * 32 * 16,), jnp.float32),
        pltpu.SemaphoreType.DMA,
        pltpu.SemaphoreType.DMA,
    ],
)(_sc_filter_body)


def _treemax(xs):
    xs = list(xs)
    while len(xs) > 1:
        nxt = [jnp.maximum(xs[k], xs[k + 1]) for k in range(0, len(xs) - 1, 2)]
        if len(xs) % 2:
            nxt.append(xs[-1])
        xs = nxt
    return xs[0]


def _scan_chunk(x, m_scr, s_scr):
    sl = [x[:, k * 128:(k + 1) * 128] for k in range(_C // 128)]
    gfold = _treemax(sl)
    m_c = jnp.max(gfold, axis=1, keepdims=True)
    m_old = m_scr[...]
    m_new = jnp.maximum(m_old, m_c)
    mb = m_new[:, 0:1]

    accs = [jnp.zeros((_B, 128), jnp.float32) for _ in range(4)]
    for k in range(_C // 128):
        accs[k % 4] = accs[k % 4] + jnp.exp(sl[k] - mb)
    acc = (accs[0] + accs[1]) + (accs[2] + accs[3])

    s_scr[...] = s_scr[...] * jnp.exp(m_old - m_new) + acc
    m_scr[...] = m_new


def _scan_body(x_ref, stats_ref, m_scr, s_scr):
    i = pl.program_id(0)

    @pl.when(i == 0)
    def _init():
        m_scr[...] = jnp.full((_B, 128), _NEG, jnp.float32)
        s_scr[...] = jnp.zeros((_B, 128), jnp.float32)

    @pl.when(i < _NC - 1)
    def _full():
        _scan_chunk(x_ref[...], m_scr, s_scr)

    @pl.when(i == _NC - 1)
    def _edge():
        tail = _V - (_NC - 1) * _C
        liota = jax.lax.broadcasted_iota(jnp.int32, (_B, _C), 1)
        x = jnp.where(liota < tail, x_ref[...], _NEG)
        _scan_chunk(x, m_scr, s_scr)
        l128 = jax.lax.broadcasted_iota(jnp.int32, (_B, 128), 1)
        s_b = jnp.sum(s_scr[...], axis=1, keepdims=True)
        stats_ref[...] = jnp.where(l128 == 0, m_scr[...],
                                   jnp.where(l128 == 1, s_b, 0.0))


def _select_body(gm_ref, xt_ref, ids_ref):
    r = gm_ref[...]
    gmax = jnp.max(r, axis=3)
    lt = jax.lax.broadcasted_iota(jnp.int32, (_B, _G), 1)
    tmax = jnp.max(jnp.where(lt < _V - (_NG - 1) * _G, xt_ref[...], _NEG),
                   axis=1, keepdims=True)
    w_i = jax.lax.broadcasted_iota(jnp.int32, (32, 32), 0)
    gs = jax.lax.broadcasted_iota(jnp.int32, (32, 32), 1)
    gid = w_i * _WG + gs
    valid = (gs < _WG) & (gid < _NG - 1)
    tail_slot = (w_i == 31) & (gs == 15)
    lane8 = jax.lax.broadcasted_iota(jnp.int32, (1, _B), 1)
    id_rows = []
    for bb in range(_B):
        sub = jnp.where(valid, gmax[bb], _NEG)
        sub = jnp.where(tail_slot, tmax[bb:bb + 1, 0:1], sub)
        idr = jnp.zeros((1, _B), jnp.int32)
        for j in range(_B):
            v = jnp.max(sub)
            sel = jnp.min(jnp.where(sub >= v, gid, _IBIG))
            idr = jnp.where(lane8 == j, sel, idr)
            sub = jnp.where(gid == sel, _NEG, sub)
        id_rows.append(idr)
    ids_ref[...] = jnp.concatenate(id_rows, axis=0)


def _merge_body(ids_ref, *refs):
    xrefs = refs[:_B * _B]
    stats_ref, energy_ref, hid_ref = refs[_B * _B:_B * _B + 3]
    act_ref, logp_ref, eng_ref, tok_ref, hid_out_ref = refs[_B * _B + 3:]

    li = jax.lax.broadcasted_iota(jnp.int32, (1, _G), 1)
    rows = []
    lidx_rows = []
    for b in range(_B):
        rows.append(jnp.concatenate(
            [xrefs[b * _B + j][b:b + 1, :] for j in range(_B)], axis=1))
        lidx_rows.append(jnp.concatenate(
            [ids_ref[b * _B + j] * _G + li for j in range(_B)], axis=1))
    cand = jnp.concatenate(rows, axis=0)
    lidx = jnp.concatenate(lidx_rows, axis=0)
    cand = jnp.where(lidx < _V, cand, _NEG)

    lane8 = jax.lax.broadcasted_iota(jnp.int32, (1, _B), 1)
    topv = jnp.zeros((_B, _B), jnp.float32)
    topi = jnp.zeros((_B, _B), jnp.int32)
    for j in range(_B):
        v = jnp.max(cand, axis=1, keepdims=True)
        sel = jnp.min(jnp.where(cand >= v, lidx, _IBIG),
                      axis=1, keepdims=True)
        topv = jnp.where(lane8 == j, v, topv)
        topi = jnp.where(lane8 == j, sel, topi)
        cand = jnp.where(lidx == sel, _NEG, cand)

    stats = stats_ref[...]
    m = stats[:, 0:1]
    s = stats[:, 1:2]
    energy = energy_ref[...]
    cand_logp = (topv - m) - jnp.log(s)
    action = energy - cand_logp

    cidx = (jax.lax.broadcasted_iota(jnp.int32, (_B, _B), 0) * _B
            + jax.lax.broadcasted_iota(jnp.int32, (_B, _B), 1))
    row8 = jax.lax.broadcasted_iota(jnp.int32, (_B, _B), 0)
    col8 = jax.lax.broadcasted_iota(jnp.int32, (_B, _B), 1)

    a_acc = jnp.zeros((1, _B), jnp.float32)
    lp_acc = jnp.zeros((1, _B), jnp.float32)
    en_acc = jnp.zeros((1, _B), jnp.float32)
    tk_acc = jnp.zeros((1, _B), jnp.int32)
    onehot = jnp.zeros((_B, _B), jnp.float32)
    energy_b = jnp.broadcast_to(energy, (_B, _B))

    aw = action
    for j in range(_B):
        a = jnp.min(aw)
        sel = jnp.min(jnp.where(aw <= a, cidx, _IBIG))
        hit = cidx == sel
        a_acc = jnp.where(lane8 == j, a, a_acc)
        lp_acc = jnp.where(lane8 == j,
                           jnp.sum(jnp.where(hit, cand_logp, 0.0)), lp_acc)
        en_acc = jnp.where(lane8 == j,
                           jnp.sum(jnp.where(hit, energy_b, 0.0)), en_acc)
        tk_acc = jnp.where(lane8 == j, jnp.sum(jnp.where(hit, topi, 0)),
                           tk_acc)
        beam = sel // _B
        onehot = onehot + jnp.where((row8 == j) & (col8 == beam), 1.0, 0.0)
        aw = jnp.where(hit, jnp.inf, aw)

    act_ref[...] = a_acc
    logp_ref[...] = lp_acc
    eng_ref[...] = en_acc
    tok_ref[...] = tk_acc
    hid_out_ref[...] = jax.lax.dot(onehot, hid_ref[...],
                                   preferred_element_type=jnp.float32)


@functools.partial(jax.jit, static_argnames=("interpret",))
def _impl(next_token_logits, next_hidden_states, energies, interpret=False):
    gm_flat = _sc_filter(next_token_logits)
    gm4 = gm_flat.reshape(_B, 32, 32, 16)

    stats = pl.pallas_call(
        _scan_body,
        grid=(_NC,),
        in_specs=[pl.BlockSpec((_B, _C), lambda i: (0, i))],
        out_specs=pl.BlockSpec((_B, 128), lambda i: (0, 0)),
        out_shape=jax.ShapeDtypeStruct((_B, 128), jnp.float32),
        scratch_shapes=[
            pltpu.VMEM((_B, 128), jnp.float32),
            pltpu.VMEM((_B, 128), jnp.float32),
        ],
        interpret=interpret,
    )(next_token_logits)

    ids = pl.pallas_call(
        _select_body,
        grid=(1,),
        in_specs=[
            pl.BlockSpec((_B, 32, 32, 16), lambda i: (0, 0, 0, 0)),
            pl.BlockSpec((_B, _G), lambda i: (0, _NG - 1)),
        ],
        out_specs=pl.BlockSpec((_B, _B), lambda i: (0, 0)),
        out_shape=jax.ShapeDtypeStruct((_B, _B), jnp.int32),
        interpret=interpret,
    )(gm4, next_token_logits)

    def _xmap(r):
        return lambda i, ids_pf: (0, ids_pf[r])

    grid_spec = pltpu.PrefetchScalarGridSpec(
        num_scalar_prefetch=1,
        grid=(1,),
        in_specs=(
            [pl.BlockSpec((_B, _G), _xmap(r)) for r in range(_B * _B)]
            + [
                pl.BlockSpec((_B, 128), lambda i, ids_pf: (0, 0)),
                pl.BlockSpec((_B, 1), lambda i, ids_pf: (0, 0)),
                pl.BlockSpec((_B, 4096), lambda i, ids_pf: (0, 0)),
            ]),
        out_specs=[
            pl.BlockSpec((1, _B), lambda i, ids_pf: (0, 0)),
            pl.BlockSpec((1, _B), lambda i, ids_pf: (0, 0)),
            pl.BlockSpec((1, _B), lambda i, ids_pf: (0, 0)),
            pl.BlockSpec((1, _B), lambda i, ids_pf: (0, 0)),
            pl.BlockSpec((_B, 4096), lambda i, ids_pf: (0, 0)),
        ],
    )
    act, logp, eng, tok, hid = pl.pallas_call(
        _merge_body,
        grid_spec=grid_spec,
        out_shape=[
            jax.ShapeDtypeStruct((1, _B), jnp.float32),
            jax.ShapeDtypeStruct((1, _B), jnp.float32),
            jax.ShapeDtypeStruct((1, _B), jnp.float32),
            jax.ShapeDtypeStruct((1, _B), jnp.int32),
            jax.ShapeDtypeStruct((_B, 4096), jnp.float32),
        ],
        interpret=interpret,
    )(ids.reshape(_B * _B), *([next_token_logits] * (_B * _B)),
      stats, energies.reshape(_B, 1), next_hidden_states)

    return (act.reshape(_B), logp.reshape(_B), eng.reshape(_B),
            tok.reshape(_B), hid)


def kernel(next_token_logits, next_hidden_states, energies):
    return _impl(next_token_logits, next_hidden_states, energies)


# speedup vs baseline: 1.0606x; 1.0019x over previous
import functools

import jax
import jax.numpy as jnp
from jax import lax
from jax.experimental import pallas as pl
from jax.experimental.pallas import tpu as pltpu
from jax.experimental.pallas import tpu_sc as plsc

_B = 8
_V = 1000000
_C = 65536
_NC = 16
_G = 1024
_NG = 977
_NEG = -jnp.inf
_IBIG = 2**30

_WG = 31
_WL = _WG * _G


def _sc_group_max(buf, obuf, b, g_chunk, oslot, n_vregs):
    na = 8
    accs = [buf[b, pl.ds(g_chunk * _G + k * 16, 16)] for k in range(na)]
    for k in range(na, n_vregs):
        accs[k % na] = jnp.maximum(accs[k % na],
                                   buf[b, pl.ds(g_chunk * _G + k * 16, 16)])
    while len(accs) > 1:
        accs = [jnp.maximum(accs[k], accs[k + 1])
                for k in range(0, len(accs), 2)]
    obuf[pl.ds(oslot * 16, 16)] = accs[0]


def _sc_filter_body(x_hbm, out_hbm, buf0, buf1, obuf, sem0, sem1):
    wid = lax.axis_index("s") * 2 + lax.axis_index("c")
    start = wid * _WL
    bufs = [buf0, buf1]
    sems = [sem0, sem1]

    def make_loop(buf, goff, ng):
        def body(t, carry):
            b = t & 7
            g = t >> 3
            _sc_group_max(buf, obuf, b, g, b * 32 + goff + g, 64)
            return carry
        return body

    def run(sizes):
        offs = [sum(sizes[:k]) for k in range(len(sizes))]
        n = len(sizes)

        def issue(ch):
            slot = ch % 2
            return pltpu.async_copy(
                x_hbm.at[:, pl.ds(start + offs[ch] * _G, sizes[ch] * _G)],
                bufs[slot].at[:, pl.ds(0, sizes[ch] * _G)], sems[slot])

        copies = {}
        for ch in range(min(2, n)):
            copies[ch] = issue(ch)
        for ch in range(n):
            copies[ch].wait()
            lax.fori_loop(0, 8 * sizes[ch],
                          make_loop(bufs[ch % 2], offs[ch], sizes[ch]), 0)
            if ch + 2 < n:
                copies[ch + 2] = issue(ch + 2)

    @pl.when(wid < 31)
    def _main():
        run([6, 6, 6, 6, 6, 1])

    @pl.when(wid == 31)
    def _tail():
        run([6, 6, 3])

    for b in range(_B):
        pltpu.sync_copy(obuf.at[pl.ds(b * 512, 512)],
                        out_hbm.at[pl.ds((b * 32 + wid) * 512, 512)])


_sc_filter = functools.partial(
    pl.kernel,
    out_type=jax.ShapeDtypeStruct((_B * 32 * 512,), jnp.float32),
    mesh=plsc.VectorSubcoreMesh(core_axis_name="c", subcore_axis_name="s",
                                num_cores=2, num_subcores=16),
    scratch_types=[
        pltpu.VMEM((_B, 6 * _G), jnp.float32),
        pltpu.VMEM((_B, 6 * _G), jnp.float32),
        pltpu.VMEM((_B * 32 * 16,), jnp.float32),
        pltpu.SemaphoreType.DMA,
        pltpu.SemaphoreType.DMA,
    ],
)(_sc_filter_body)


def _treemax(xs):
    xs = list(xs)
    while len(xs) > 1:
        nxt = [jnp.maximum(xs[k], xs[k + 1]) for k in range(0, len(xs) - 1, 2)]
        if len(xs) % 2:
            nxt.append(xs[-1])
        xs = nxt
    return xs[0]


def _scan_chunk(x, m_scr, s_scr):
    sl = [x[:, k * 128:(k + 1) * 128] for k in range(_C // 128)]
    gfold = _treemax(sl)
    m_c = jnp.max(gfold, axis=1, keepdims=True)
    m_old = m_scr[...]
    m_new = jnp.maximum(m_old, m_c)
    mb = m_new[:, 0:1]

    accs = [jnp.zeros((_B, 128), jnp.float32) for _ in range(4)]
    for k in range(_C // 128):
        accs[k % 4] = accs[k % 4] + jnp.exp(sl[k] - mb)
    acc = (accs[0] + accs[1]) + (accs[2] + accs[3])

    s_scr[...] = s_scr[...] * jnp.exp(m_old - m_new) + acc
    m_scr[...] = m_new


def _scan_body(x_ref, stats_ref, m_scr, s_scr):
    i = pl.program_id(0)

    @pl.when(i == 0)
    def _init():
        m_scr[...] = jnp.full((_B, 128), _NEG, jnp.float32)
        s_scr[...] = jnp.zeros((_B, 128), jnp.float32)

    @pl.when(i < _NC - 1)
    def _full():
        _scan_chunk(x_ref[...], m_scr, s_scr)

    @pl.when(i == _NC - 1)
    def _edge():
        tail = _V - (_NC - 1) * _C
        liota = jax.lax.broadcasted_iota(jnp.int32, (_B, _C), 1)
        x = jnp.where(liota < tail, x_ref[...], _NEG)
        _scan_chunk(x, m_scr, s_scr)
        l128 = jax.lax.broadcasted_iota(jnp.int32, (_B, 128), 1)
        s_b = jnp.sum(s_scr[...], axis=1, keepdims=True)
        stats_ref[...] = jnp.where(l128 == 0, m_scr[...],
                                   jnp.where(l128 == 1, s_b, 0.0))


def _select_body(gm_ref, xt_ref, ids_ref):
    r = gm_ref[...]
    gmax = jnp.max(r, axis=3)
    lt = jax.lax.broadcasted_iota(jnp.int32, (_B, _G), 1)
    tmax = jnp.max(jnp.where(lt < _V - (_NG - 1) * _G, xt_ref[...], _NEG),
                   axis=1, keepdims=True)
    w_i = jax.lax.broadcasted_iota(jnp.int32, (32, 32), 0)
    gs = jax.lax.broadcasted_iota(jnp.int32, (32, 32), 1)
    gid = w_i * _WG + gs
    valid = (gs < _WG) & (gid < _NG - 1)
    tail_slot = (w_i == 31) & (gs == 15)
    lane8 = jax.lax.broadcasted_iota(jnp.int32, (1, _B), 1)
    id_rows = []
    for bb in range(_B):
        sub = jnp.where(valid, gmax[bb], _NEG)
        sub = jnp.where(tail_slot, tmax[bb:bb + 1, 0:1], sub)
        idr = jnp.zeros((1, _B), jnp.int32)
        for j in range(_B):
            v = jnp.max(sub)
            sel = jnp.min(jnp.where(sub >= v, gid, _IBIG))
            idr = jnp.where(lane8 == j, sel, idr)
            sub = jnp.where(gid == sel, _NEG, sub)
        id_rows.append(idr)
    ids_ref[...] = jnp.concatenate(id_rows, axis=0)


def _merge_body(ids_ref, *refs):
    xrefs = refs[:_B * _B]
    stats_ref, energy_ref, hid_ref = refs[_B * _B:_B * _B + 3]
    act_ref, logp_ref, eng_ref, tok_ref, hid_out_ref = refs[_B * _B + 3:]

    li = jax.lax.broadcasted_iota(jnp.int32, (1, _G), 1)
    rows = []
    lidx_rows = []
    for b in range(_B):
        rows.append(jnp.concatenate(
            [xrefs[b * _B + j][b:b + 1, :] for j in range(_B)], axis=1))
        lidx_rows.append(jnp.concatenate(
            [ids_ref[b * _B + j] * _G + li for j in range(_B)], axis=1))
    cand = jnp.concatenate(rows, axis=0)
    lidx = jnp.concatenate(lidx_rows, axis=0)
    cand = jnp.where(lidx < _V, cand, _NEG)

    lane8 = jax.lax.broadcasted_iota(jnp.int32, (1, _B), 1)
    topv = jnp.zeros((_B, _B), jnp.float32)
    topi = jnp.zeros((_B, _B), jnp.int32)
    for j in range(_B):
        v = jnp.max(cand, axis=1, keepdims=True)
        sel = jnp.min(jnp.where(cand >= v, lidx, _IBIG),
                      axis=1, keepdims=True)
        topv = jnp.where(lane8 == j, v, topv)
        topi = jnp.where(lane8 == j, sel, topi)
        cand = jnp.where(lidx == sel, _NEG, cand)

    stats = stats_ref[...]
    m = stats[:, 0:1]
    s = stats[:, 1:2]
    energy = energy_ref[...]
    cand_logp = (topv - m) - jnp.log(s)
    action = energy - cand_logp

    cidx = (jax.lax.broadcasted_iota(jnp.int32, (_B, _B), 0) * _B
            + jax.lax.broadcasted_iota(jnp.int32, (_B, _B), 1))
    row8 = jax.lax.broadcasted_iota(jnp.int32, (_B, _B), 0)
    col8 = jax.lax.broadcasted_iota(jnp.int32, (_B, _B), 1)

    a_acc = jnp.zeros((1, _B), jnp.float32)
    lp_acc = jnp.zeros((1, _B), jnp.float32)
    en_acc = jnp.zeros((1, _B), jnp.float32)
    tk_acc = jnp.zeros((1, _B), jnp.int32)
    onehot = jnp.zeros((_B, _B), jnp.float32)
    energy_b = jnp.broadcast_to(energy, (_B, _B))

    aw = action
    for j in range(_B):
        a = jnp.min(aw)
        sel = jnp.min(jnp.where(aw <= a, cidx, _IBIG))
        hit = cidx == sel
        a_acc = jnp.where(lane8 == j, a, a_acc)
        lp_acc = jnp.where(lane8 == j,
                           jnp.sum(jnp.where(hit, cand_logp, 0.0)), lp_acc)
        en_acc = jnp.where(lane8 == j,
                           jnp.sum(jnp.where(hit, energy_b, 0.0)), en_acc)
        tk_acc = jnp.where(lane8 == j, jnp.sum(jnp.where(hit, topi, 0)),
                           tk_acc)
        beam = sel // _B
        onehot = onehot + jnp.where((row8 == j) & (col8 == beam), 1.0, 0.0)
        aw = jnp.where(hit, jnp.inf, aw)

    act_ref[...] = a_acc
    logp_ref[...] = lp_acc
    eng_ref[...] = en_acc
    tok_ref[...] = tk_acc
    hid_out_ref[...] = jax.lax.dot(onehot, hid_ref[...],
                                   preferred_element_type=jnp.float32)


@functools.partial(jax.jit, static_argnames=("interpret",))
def _impl(next_token_logits, next_hidden_states, energies, interpret=False):
    gm_flat = _sc_filter(next_token_logits)
    gm4 = gm_flat.reshape(_B, 32, 32, 16)

    stats = pl.pallas_call(
        _scan_body,
        grid=(_NC,),
        in_specs=[pl.BlockSpec((_B, _C), lambda i: (0, i))],
        out_specs=pl.BlockSpec((_B, 128), lambda i: (0, 0)),
        out_shape=jax.ShapeDtypeStruct((_B, 128), jnp.float32),
        scratch_shapes=[
            pltpu.VMEM((_B, 128), jnp.float32),
            pltpu.VMEM((_B, 128), jnp.float32),
        ],
        interpret=interpret,
    )(next_token_logits)

    ids = pl.pallas_call(
        _select_body,
        grid=(1,),
        in_specs=[
            pl.BlockSpec((_B, 32, 32, 16), lambda i: (0, 0, 0, 0)),
            pl.BlockSpec((_B, _G), lambda i: (0, _NG - 1)),
        ],
        out_specs=pl.BlockSpec((_B, _B), lambda i: (0, 0)),
        out_shape=jax.ShapeDtypeStruct((_B, _B), jnp.int32),
        interpret=interpret,
    )(gm4, next_token_logits)

    def _xmap(r):
        return lambda i, ids_pf: (0, ids_pf[r])

    grid_spec = pltpu.PrefetchScalarGridSpec(
        num_scalar_prefetch=1,
        grid=(1,),
        in_specs=(
            [pl.BlockSpec((_B, _G), _xmap(r)) for r in range(_B * _B)]
            + [
                pl.BlockSpec((_B, 128), lambda i, ids_pf: (0, 0)),
                pl.BlockSpec((_B, 1), lambda i, ids_pf: (0, 0)),
                pl.BlockSpec((_B, 4096), lambda i, ids_pf: (0, 0)),
            ]),
        out_specs=[
            pl.BlockSpec((1, _B), lambda i, ids_pf: (0, 0)),
            pl.BlockSpec((1, _B), lambda i, ids_pf: (0, 0)),
            pl.BlockSpec((1, _B), lambda i, ids_pf: (0, 0)),
            pl.BlockSpec((1, _B), lambda i, ids_pf: (0, 0)),
            pl.BlockSpec((_B, 4096), lambda i, ids_pf: (0, 0)),
        ],
    )
    act, logp, eng, tok, hid = pl.pallas_call(
        _merge_body,
        grid_spec=grid_spec,
        out_shape=[
            jax.ShapeDtypeStruct((1, _B), jnp.float32),
            jax.ShapeDtypeStruct((1, _B), jnp.float32),
            jax.ShapeDtypeStruct((1, _B), jnp.float32),
            jax.ShapeDtypeStruct((1, _B), jnp.int32),
            jax.ShapeDtypeStruct((_B, 4096), jnp.float32),
        ],
        interpret=interpret,
    )(ids.reshape(_B * _B), *([next_token_logits] * (_B * _B)),
      stats, energies.reshape(_B, 1), next_hidden_states)

    return (act.reshape(_B), logp.reshape(_B), eng.reshape(_B),
            tok.reshape(_B), hid)


def kernel(next_token_logits, next_hidden_states, energies):
    return _impl(next_token_logits, next_hidden_states, energies)
